# baseline (speedup 1.0000x reference)
import functools
import jax
import jax.numpy as jnp
from jax import lax
from jax.scipy.linalg import block_diag as _block_diag
from jax.experimental import pallas as pl
from jax.experimental.pallas import tpu as pltpu
from jax.experimental.pallas import tpu_sc as plsc

EMBED = 32
CHUNK = 128
NW = 32
C = 4


def _sc_body(ut4_hbm, mt4_hbm, idx_hbm, ue_hbm, me_hbm,
             idx_v, sidx_v, slab_v, oue_v, ome_v, sem0, sem1):
    wid = lax.axis_index("s") * 2 + lax.axis_index("c")
    pltpu.sync_copy(idx_hbm.at[wid], idx_v)
    lane = jnp.arange(16, dtype=jnp.int32)

    for k in range(2 * C):
        for g in range(CHUNK // 16):
            sidx_v[k, pl.ds(g * 16, 16)] = idx_v[k, pl.ds(g * 16, 16)] >> 2

    sems = (sem0, sem1)
    tables = (ut4_hbm, mt4_hbm)

    def start(k):
        return pltpu.async_copy(
            tables[k // C].at[sidx_v.at[k]], slab_v.at[k % 2], sems[k % 2])

    def extract(k):
        slab = slab_v.at[k % 2]
        out = oue_v if k < C else ome_v
        j = k % C
        jvecs, colbs, e0s = [], [], []
        for g in range(CHUNK // 16):
            iv = idx_v[k, pl.ds(g * 16, 16)]
            jvecs.append(lane + g * 16)
            colbs.append((iv & 3) * EMBED)
            e0s.append((j * CHUNK + g * 16 + lane) * EMBED)

        def body(ci, carry):
            for g in range(CHUNK // 16):
                vals = plsc.load_gather(slab, [jvecs[g], colbs[g] + ci])
                e = e0s[g] + ci
                plsc.store_scatter(out, [e >> 7, e & 127], vals)
            return carry
        lax.fori_loop(0, EMBED, body, 0)

    cps = {0: start(0)}
    for k in range(2 * C):
        if k + 1 < 2 * C:
            cps[k + 1] = start(k + 1)
        cps[k].wait()
        extract(k)

    pltpu.sync_copy(oue_v, ue_hbm.at[wid])
    pltpu.sync_copy(ome_v, me_hbm.at[wid])


def _make_sc_gather():
    mesh = plsc.VectorSubcoreMesh(core_axis_name="c", subcore_axis_name="s")
    return pl.kernel(
        _sc_body,
        mesh=mesh,
        compiler_params=pltpu.CompilerParams(needs_layout_passes=False),
        out_type=(
            jax.ShapeDtypeStruct((NW, C * EMBED, 128), jnp.float32),
            jax.ShapeDtypeStruct((NW, C * EMBED, 128), jnp.float32),
        ),
        scratch_types=[
            pltpu.VMEM((2 * C, CHUNK), jnp.int32),
            pltpu.VMEM((2 * C, CHUNK), jnp.int32),
            pltpu.VMEM((2, CHUNK, 128), jnp.float32),
            pltpu.VMEM((C * EMBED, 128), jnp.float32),
            pltpu.VMEM((C * EMBED, 128), jnp.float32),
            pltpu.SemaphoreType.DMA,
            pltpu.SemaphoreType.DMA,
        ],
    )


def _mlp_body(u4_ref, m4_ref, W1u_ref, W1m_ref, b1_ref, W2_ref, b2_ref,
              W3_ref, b3_ref, o_ref):
    x = (jnp.dot(u4_ref[...], W1u_ref[...], preferred_element_type=jnp.float32)
         + jnp.dot(m4_ref[...], W1m_ref[...], preferred_element_type=jnp.float32)
         + b1_ref[...])
    x = jnp.where(x >= 0, x, 0.01 * x)
    x = jnp.dot(x, W2_ref[...], preferred_element_type=jnp.float32) + b2_ref[...]
    x = jnp.where(x >= 0, x, 0.01 * x)
    o_ref[...] = jnp.dot(x, W3_ref[...], preferred_element_type=jnp.float32) + b3_ref[...]


def _mlp(u4, m4, W1u4, W1m4, b14, W24, b24, W34, b34, BT4):
    B4 = u4.shape[0]
    grid = (B4 // BT4,)
    return pl.pallas_call(
        _mlp_body,
        grid=grid,
        in_specs=[
            pl.BlockSpec((BT4, 128), lambda i: (i, 0)),
            pl.BlockSpec((BT4, 128), lambda i: (i, 0)),
            pl.BlockSpec((128, 512), lambda i: (0, 0)),
            pl.BlockSpec((128, 512), lambda i: (0, 0)),
            pl.BlockSpec((1, 512), lambda i: (0, 0)),
            pl.BlockSpec((512, 1024), lambda i: (0, 0)),
            pl.BlockSpec((1, 1024), lambda i: (0, 0)),
            pl.BlockSpec((1024, 4), lambda i: (0, 0)),
            pl.BlockSpec((1, 4), lambda i: (0, 0)),
        ],
        out_specs=pl.BlockSpec((BT4, 4), lambda i: (i, 0)),
        out_shape=jax.ShapeDtypeStruct((B4, 4), jnp.float32),
    )(u4, m4, W1u4, W1m4, b14.reshape(1, -1), W24, b24.reshape(1, -1),
      W34, b34.reshape(1, -1))


def kernel(user, movie, user_table, movie_table, W1, b1, W2, b2, W3, b3):
    B = user.shape[0]
    ut4 = user_table.reshape(-1, 128)
    mt4 = movie_table.reshape(-1, 128)
    uidx = user.astype(jnp.int32).reshape(NW, C, CHUNK)
    midx = movie.astype(jnp.int32).reshape(NW, C, CHUNK)
    allidx = jnp.concatenate([uidx, midx], axis=1)
    ue, me = _make_sc_gather()(ut4, mt4, allidx)
    u4 = ue.reshape(B // 4, 128)
    m4 = me.reshape(B // 4, 128)
    W1u4 = _block_diag(*([W1[:EMBED, :]] * 4))
    W1m4 = _block_diag(*([W1[EMBED:, :]] * 4))
    W24 = _block_diag(*([W2] * 4))
    W34 = _block_diag(*([W3] * 4))
    b14 = jnp.tile(b1, 4)
    b24 = jnp.tile(b2, 4)
    b34 = jnp.tile(b3, 4)
    out4 = _mlp(u4, m4, W1u4, W1m4, b14, W24, b24, W34, b34, BT4=512)
    return out4.reshape(B, 1)


# speedup vs baseline: 1.3710x; 1.3710x over previous
import functools
import jax
import jax.numpy as jnp
from jax import lax
from jax.scipy.linalg import block_diag as _block_diag
from jax.experimental import pallas as pl
from jax.experimental.pallas import tpu as pltpu
from jax.experimental.pallas import tpu_sc as plsc

EMBED = 32
NW = 32
WSZ = 16
NWAVE = 32


def _sc_body(ut_hbm, mt_hbm, idx_hbm, out_hbm, idx_v, ring_v, out_v, sem0, sem1):
    sems = (sem0, sem1)
    wid = lax.axis_index("s") * 2 + lax.axis_index("c")
    pltpu.sync_copy(idx_hbm.at[wid], idx_v)
    lane = jnp.arange(16, dtype=jnp.int32)

    def wave_idx(w):
        return idx_v[w // 8, pl.ds((w % 8) * WSZ, WSZ)]

    def start(tbl, w, s):
        iv = wave_idx(w)
        for j in range(WSZ):
            sj = jnp.sum(jnp.where(lane == j, iv, 0))
            base8 = pl.multiple_of((sj >> 3) * 8, 8)
            pltpu.make_async_copy(
                tbl.at[pl.ds(base8, 8), :], ring_v.at[s, j], sems[s]).start()

    def drain(tbl, s):
        for _ in range(WSZ):
            pltpu.make_async_copy(
                tbl.at[pl.ds(0, 8), :], ring_v.at[s, 0], sems[s]).wait()

    def extract(w, s):
        iv = wave_idx(w)
        iv7 = iv & 7
        e0 = (w * WSZ + lane) * EMBED
        slab = ring_v.at[s]

        def ebody(ci, carry):
            vals = plsc.load_gather(slab, [lane, iv7, iv7 * 0 + ci])
            e = e0 + ci
            plsc.store_scatter(out_v, [e >> 7, e & 127], vals)
            return carry
        lax.fori_loop(0, EMBED, ebody, 0)

    def table_pass(tbl, woff, guard_last):
        start(tbl, woff, 0)

        def body(p, carry):
            w0 = woff + 2 * p
            start(tbl, w0 + 1, 1)
            drain(tbl, 0)
            extract(w0, 0)

            @pl.when(p < NWAVE // 2 - 1)
            def _():
                start(tbl, w0 + 2, 0)
            drain(tbl, 1)
            extract(w0 + 1, 1)
            return carry
        lax.fori_loop(0, NWAVE // 2, body, 0)

    table_pass(ut_hbm, 0, True)
    table_pass(mt_hbm, NWAVE, True)
    pltpu.sync_copy(out_v, out_hbm.at[wid])


def _make_sc_gather():
    mesh = plsc.VectorSubcoreMesh(core_axis_name="c", subcore_axis_name="s")
    return pl.kernel(
        _sc_body,
        mesh=mesh,
        compiler_params=pltpu.CompilerParams(needs_layout_passes=False),
        out_type=jax.ShapeDtypeStruct((NW, 256, 128), jnp.float32),
        scratch_types=[
            pltpu.VMEM((8, 128), jnp.int32),
            pltpu.VMEM((2, WSZ, 8, EMBED), jnp.float32),
            pltpu.VMEM((256, 128), jnp.float32),
            pltpu.SemaphoreType.DMA,
            pltpu.SemaphoreType.DMA,
        ],
    )


def _mlp_body(u4_ref, m4_ref, W1u_ref, W1m_ref, b1_ref, W2_ref, b2_ref,
              W3_ref, b3_ref, o_ref):
    x = (jnp.dot(u4_ref[...], W1u_ref[...], preferred_element_type=jnp.float32)
         + jnp.dot(m4_ref[...], W1m_ref[...], preferred_element_type=jnp.float32)
         + b1_ref[...])
    x = jnp.where(x >= 0, x, 0.01 * x)
    x = jnp.dot(x, W2_ref[...], preferred_element_type=jnp.float32) + b2_ref[...]
    x = jnp.where(x >= 0, x, 0.01 * x)
    o_ref[...] = jnp.dot(x, W3_ref[...], preferred_element_type=jnp.float32) + b3_ref[...]


def _mlp(x, W1u4, W1m4, b14, W24, b24, W34, b34):
    grid = (NW,)
    return pl.pallas_call(
        _mlp_body,
        grid=grid,
        in_specs=[
            pl.BlockSpec((128, 128), lambda i: (2 * i, 0)),
            pl.BlockSpec((128, 128), lambda i: (2 * i + 1, 0)),
            pl.BlockSpec((128, 512), lambda i: (0, 0)),
            pl.BlockSpec((128, 512), lambda i: (0, 0)),
            pl.BlockSpec((1, 512), lambda i: (0, 0)),
            pl.BlockSpec((512, 1024), lambda i: (0, 0)),
            pl.BlockSpec((1, 1024), lambda i: (0, 0)),
            pl.BlockSpec((1024, 4), lambda i: (0, 0)),
            pl.BlockSpec((1, 4), lambda i: (0, 0)),
        ],
        out_specs=pl.BlockSpec((128, 4), lambda i: (i, 0)),
        out_shape=jax.ShapeDtypeStruct((NW * 128, 4), jnp.float32),
    )(x, x, W1u4, W1m4, b14.reshape(1, -1), W24, b24.reshape(1, -1),
      W34, b34.reshape(1, -1))


def kernel(user, movie, user_table, movie_table, W1, b1, W2, b2, W3, b3):
    B = user.shape[0]
    uidx = user.astype(jnp.int32).reshape(NW, 4, 128)
    midx = movie.astype(jnp.int32).reshape(NW, 4, 128)
    allidx = jnp.concatenate([uidx, midx], axis=1)
    packed = _make_sc_gather()(user_table, movie_table, allidx)
    x = packed.reshape(NW * 256, 128)
    W1u4 = _block_diag(*([W1[:EMBED, :]] * 4))
    W1m4 = _block_diag(*([W1[EMBED:, :]] * 4))
    W24 = _block_diag(*([W2] * 4))
    W34 = _block_diag(*([W3] * 4))
    b14 = jnp.tile(b1, 4)
    b24 = jnp.tile(b2, 4)
    b34 = jnp.tile(b3, 4)
    out4 = _mlp(x, W1u4, W1m4, b14, W24, b24, W34, b34)
    return out4.reshape(B, 1)


# speedup vs baseline: 1.4486x; 1.0566x over previous
import functools
import jax
import jax.numpy as jnp
from jax import lax
from jax.scipy.linalg import block_diag as _block_diag
from jax.experimental import pallas as pl
from jax.experimental.pallas import tpu as pltpu
from jax.experimental.pallas import tpu_sc as plsc

EMBED = 32
NW = 32
WSZ = 16
NWAVE = 32


def _sc_body(tbl_hbm, idx_hbm, out_hbm, idx_v, ring_v, out_v, sem0, sem1):
    sems = (sem0, sem1)
    wid = lax.axis_index("s") * 2 + lax.axis_index("c")
    pltpu.sync_copy(idx_hbm.at[wid], idx_v)
    lane = jnp.arange(16, dtype=jnp.int32)

    def wave_idx(w):
        return idx_v[w // 8, pl.ds((w % 8) * WSZ, WSZ)]

    def start(tbl, w, s):
        iv = wave_idx(w)
        for j in range(WSZ):
            sj = jnp.sum(jnp.where(lane == j, iv, 0))
            base8 = pl.multiple_of((sj >> 3) * 8, 8)
            pltpu.make_async_copy(
                tbl.at[pl.ds(base8, 8), :], ring_v.at[s, j], sems[s]).start()

    def drain(tbl, s):
        for _ in range(WSZ):
            pltpu.make_async_copy(
                tbl.at[pl.ds(0, 8), :], ring_v.at[s, 0], sems[s]).wait()

    def extract(w, s):
        iv = wave_idx(w)
        iv7 = iv & 7
        e0 = (w * WSZ + lane) * EMBED
        slab = ring_v.at[s]

        def ebody(ci, carry):
            vals = plsc.load_gather(slab, [lane, iv7, iv7 * 0 + ci])
            e = e0 + ci
            plsc.store_scatter(out_v, [e >> 7, e & 127], vals)
            return carry
        lax.fori_loop(0, EMBED, ebody, 0)

    def table_pass(tbl, woff, guard_last):
        start(tbl, woff, 0)

        def body(p, carry):
            w0 = woff + 2 * p
            start(tbl, w0 + 1, 1)
            drain(tbl, 0)
            extract(w0, 0)

            @pl.when(p < NWAVE // 2 - 1)
            def _():
                start(tbl, w0 + 2, 0)
            drain(tbl, 1)
            extract(w0 + 1, 1)
            return carry
        lax.fori_loop(0, NWAVE // 2, body, 0)

    table_pass(tbl_hbm, 0, True)
    pltpu.sync_copy(out_v, out_hbm.at[wid])


def _make_sc_gather():
    mesh = plsc.VectorSubcoreMesh(core_axis_name="c", subcore_axis_name="s")
    return pl.kernel(
        _sc_body,
        mesh=mesh,
        compiler_params=pltpu.CompilerParams(needs_layout_passes=False),
        out_type=jax.ShapeDtypeStruct((NW, 128, 128), jnp.float32),
        scratch_types=[
            pltpu.VMEM((4, 128), jnp.int32),
            pltpu.VMEM((2, WSZ, 8, EMBED), jnp.float32),
            pltpu.VMEM((128, 128), jnp.float32),
            pltpu.SemaphoreType.DMA,
            pltpu.SemaphoreType.DMA,
        ],
    )


def _mlp_body(u4_ref, m4_ref, W1u_ref, W1m_ref, b1_ref, W2_ref, b2_ref,
              W3_ref, b3_ref, o_ref):
    x = (jnp.dot(u4_ref[...], W1u_ref[...], preferred_element_type=jnp.float32)
         + jnp.dot(m4_ref[...], W1m_ref[...], preferred_element_type=jnp.float32)
         + b1_ref[...])
    x = jnp.where(x >= 0, x, 0.01 * x)
    x = jnp.dot(x, W2_ref[...], preferred_element_type=jnp.float32) + b2_ref[...]
    x = jnp.where(x >= 0, x, 0.01 * x)
    o_ref[...] = jnp.dot(x, W3_ref[...], preferred_element_type=jnp.float32) + b3_ref[...]


def _mlp(xu, xm, W1u4, W1m4, b14, W24, b24, W34, b34):
    grid = (8,)
    return pl.pallas_call(
        _mlp_body,
        grid=grid,
        in_specs=[
            pl.BlockSpec((512, 128), lambda i: (i, 0)),
            pl.BlockSpec((512, 128), lambda i: (i, 0)),
            pl.BlockSpec((128, 512), lambda i: (0, 0)),
            pl.BlockSpec((128, 512), lambda i: (0, 0)),
            pl.BlockSpec((1, 512), lambda i: (0, 0)),
            pl.BlockSpec((512, 1024), lambda i: (0, 0)),
            pl.BlockSpec((1, 1024), lambda i: (0, 0)),
            pl.BlockSpec((1024, 4), lambda i: (0, 0)),
            pl.BlockSpec((1, 4), lambda i: (0, 0)),
        ],
        out_specs=pl.BlockSpec((512, 4), lambda i: (i, 0)),
        out_shape=jax.ShapeDtypeStruct((4096, 4), jnp.float32),
    )(xu, xm, W1u4, W1m4, b14.reshape(1, -1), W24, b24.reshape(1, -1),
      W34, b34.reshape(1, -1))


def kernel(user, movie, user_table, movie_table, W1, b1, W2, b2, W3, b3):
    B = user.shape[0]
    uidx = user.astype(jnp.int32).reshape(NW, 4, 128)
    midx = movie.astype(jnp.int32).reshape(NW, 4, 128)
    g = _make_sc_gather()
    pm = g(movie_table, midx)
    pu = g(user_table, uidx)
    xu = pu.reshape(NW * 128, 128)
    xm = pm.reshape(NW * 128, 128)
    W1u4 = _block_diag(*([W1[:EMBED, :]] * 4))
    W1m4 = _block_diag(*([W1[EMBED:, :]] * 4))
    W24 = _block_diag(*([W2] * 4))
    W34 = _block_diag(*([W3] * 4))
    b14 = jnp.tile(b1, 4)
    b24 = jnp.tile(b2, 4)
    b34 = jnp.tile(b3, 4)
    out4 = _mlp(xu, xm, W1u4, W1m4, b14, W24, b24, W34, b34)
    return out4.reshape(B, 1)


# speedup vs baseline: 1.4573x; 1.0060x over previous
import functools
import jax
import jax.numpy as jnp
from jax import lax
from jax.scipy.linalg import block_diag as _block_diag
from jax.experimental import pallas as pl
from jax.experimental.pallas import tpu as pltpu
from jax.experimental.pallas import tpu_sc as plsc

EMBED = 32
NW = 32
WSZ = 32
NWAVE = 16


def _sc_body(tbl_hbm, idx_hbm, out_hbm, idx_v, ring_v, out_v, sem0, sem1):
    sems = (sem0, sem1)
    wid = lax.axis_index("s") * 2 + lax.axis_index("c")
    pltpu.sync_copy(idx_hbm.at[wid], idx_v)
    lane = jnp.arange(16, dtype=jnp.int32)

    def wave_idx(w, h):
        return idx_v[w // 4, pl.ds((w % 4) * WSZ + h * 16, 16)]

    def start(tbl, w, s):
        for h in range(2):
            iv = wave_idx(w, h)
            for j in range(16):
                sj = jnp.sum(jnp.where(lane == j, iv, 0))
                base8 = pl.multiple_of((sj >> 3) * 8, 8)
                pltpu.make_async_copy(
                    tbl.at[pl.ds(base8, 8), :], ring_v.at[s, h * 16 + j],
                    sems[s]).start()

    def drain(tbl, s):
        for _ in range(WSZ):
            pltpu.make_async_copy(
                tbl.at[pl.ds(0, 8), :], ring_v.at[s, 0], sems[s]).wait()

    def extract(w, s):
        ivs = [wave_idx(w, h) for h in range(2)]
        slab = ring_v.at[s]

        def ebody(ci, carry):
            for h in range(2):
                iv7 = ivs[h] & 7
                vals = plsc.load_gather(
                    slab, [lane + h * 16, iv7, iv7 * 0 + ci])
                e = (w * WSZ + h * 16 + lane) * EMBED + ci
                plsc.store_scatter(out_v, [e >> 7, e & 127], vals)
            return carry
        lax.fori_loop(0, EMBED, ebody, 0)

    def table_pass(tbl, woff, guard_last):
        start(tbl, woff, 0)

        def body(p, carry):
            w0 = woff + 2 * p
            start(tbl, w0 + 1, 1)
            drain(tbl, 0)
            extract(w0, 0)

            @pl.when(p < NWAVE // 2 - 1)
            def _():
                start(tbl, w0 + 2, 0)
            drain(tbl, 1)
            extract(w0 + 1, 1)
            return carry
        lax.fori_loop(0, NWAVE // 2, body, 0)

    table_pass(tbl_hbm, 0, True)
    pltpu.sync_copy(out_v, out_hbm.at[wid])


def _make_sc_gather():
    mesh = plsc.VectorSubcoreMesh(core_axis_name="c", subcore_axis_name="s")
    return pl.kernel(
        _sc_body,
        mesh=mesh,
        compiler_params=pltpu.CompilerParams(needs_layout_passes=False),
        out_type=jax.ShapeDtypeStruct((NW, 128, 128), jnp.float32),
        scratch_types=[
            pltpu.VMEM((4, 128), jnp.int32),
            pltpu.VMEM((2, WSZ, 8, EMBED), jnp.float32),
            pltpu.VMEM((128, 128), jnp.float32),
            pltpu.SemaphoreType.DMA,
            pltpu.SemaphoreType.DMA,
        ],
    )


def _mlp_body(u4_ref, m4_ref, W1u_ref, W1m_ref, b1_ref, W2_ref, b2_ref,
              W3_ref, b3_ref, o_ref):
    x = (jnp.dot(u4_ref[...], W1u_ref[...], preferred_element_type=jnp.float32)
         + jnp.dot(m4_ref[...], W1m_ref[...], preferred_element_type=jnp.float32)
         + b1_ref[...])
    x = jnp.where(x >= 0, x, 0.01 * x)
    x = jnp.dot(x, W2_ref[...], preferred_element_type=jnp.float32) + b2_ref[...]
    x = jnp.where(x >= 0, x, 0.01 * x)
    o_ref[...] = jnp.dot(x, W3_ref[...], preferred_element_type=jnp.float32) + b3_ref[...]


def _mlp(xu, xm, W1u4, W1m4, b14, W24, b24, W34, b34):
    grid = (8,)
    return pl.pallas_call(
        _mlp_body,
        grid=grid,
        in_specs=[
            pl.BlockSpec((512, 128), lambda i: (i, 0)),
            pl.BlockSpec((512, 128), lambda i: (i, 0)),
            pl.BlockSpec((128, 512), lambda i: (0, 0)),
            pl.BlockSpec((128, 512), lambda i: (0, 0)),
            pl.BlockSpec((1, 512), lambda i: (0, 0)),
            pl.BlockSpec((512, 1024), lambda i: (0, 0)),
            pl.BlockSpec((1, 1024), lambda i: (0, 0)),
            pl.BlockSpec((1024, 4), lambda i: (0, 0)),
            pl.BlockSpec((1, 4), lambda i: (0, 0)),
        ],
        out_specs=pl.BlockSpec((512, 4), lambda i: (i, 0)),
        out_shape=jax.ShapeDtypeStruct((4096, 4), jnp.float32),
    )(xu, xm, W1u4, W1m4, b14.reshape(1, -1), W24, b24.reshape(1, -1),
      W34, b34.reshape(1, -1))


def kernel(user, movie, user_table, movie_table, W1, b1, W2, b2, W3, b3):
    B = user.shape[0]
    uidx = user.astype(jnp.int32).reshape(NW, 4, 128)
    midx = movie.astype(jnp.int32).reshape(NW, 4, 128)
    g = _make_sc_gather()
    pm = g(movie_table, midx)
    pu = g(user_table, uidx)
    xu = pu.reshape(NW * 128, 128)
    xm = pm.reshape(NW * 128, 128)
    W1u4 = _block_diag(*([W1[:EMBED, :]] * 4))
    W1m4 = _block_diag(*([W1[EMBED:, :]] * 4))
    W24 = _block_diag(*([W2] * 4))
    W34 = _block_diag(*([W3] * 4))
    b14 = jnp.tile(b1, 4)
    b24 = jnp.tile(b2, 4)
    b34 = jnp.tile(b3, 4)
    out4 = _mlp(xu, xm, W1u4, W1m4, b14, W24, b24, W34, b34)
    return out4.reshape(B, 1)


# speedup vs baseline: 2.6923x; 1.8475x over previous
import functools
import jax
import jax.numpy as jnp
from jax import lax
from jax.scipy.linalg import block_diag as _block_diag
from jax.experimental import pallas as pl
from jax.experimental.pallas import tpu as pltpu
from jax.experimental.pallas import tpu_sc as plsc

EMBED = 32
NW = 32
WSZ = 32
NWAVE = 16


def _sc_body(tbl_hbm, idx_hbm, out_hbm, idx_v, ring_v, out_v, sem0, sem1):
    sems = (sem0, sem1)
    wid = lax.axis_index("s") * 2 + lax.axis_index("c")
    pltpu.sync_copy(idx_hbm.at[wid], idx_v)
    lane = jnp.arange(16, dtype=jnp.int32)

    def wave_idx(w, h):
        return idx_v[w // 4, pl.ds((w % 4) * WSZ + h * 16, 16)]

    def start(tbl, w, s):
        for h in range(2):
            iv = wave_idx(w, h)
            for j in range(16):
                sj = jnp.sum(jnp.where(lane == j, iv, 0))
                base8 = pl.multiple_of((sj >> 3) * 8, 8)
                pltpu.make_async_copy(
                    tbl.at[pl.ds(base8, 8), :], ring_v.at[s, h * 16 + j],
                    sems[s]).start()

    def drain(tbl, s):
        for _ in range(WSZ):
            pltpu.make_async_copy(
                tbl.at[pl.ds(0, 8), :], ring_v.at[s, 0], sems[s]).wait()

    def extract(w, s):
        ivs = [wave_idx(w, h) for h in range(2)]
        slab = ring_v.at[s]

        def ebody(ci, carry):
            for h in range(2):
                iv7 = ivs[h] & 7
                vals = plsc.load_gather(
                    slab, [lane + h * 16, iv7, iv7 * 0 + ci])
                e = (w * WSZ + h * 16 + lane) * EMBED + ci
                plsc.store_scatter(out_v, [e >> 7, e & 127], vals)
            return carry
        lax.fori_loop(0, EMBED, ebody, 0)

    def table_pass(tbl, woff, guard_last):
        start(tbl, woff, 0)

        def body(p, carry):
            w0 = woff + 2 * p
            start(tbl, w0 + 1, 1)
            drain(tbl, 0)
            extract(w0, 0)

            @pl.when(p < NWAVE // 2 - 1)
            def _():
                start(tbl, w0 + 2, 0)
            drain(tbl, 1)
            extract(w0 + 1, 1)
            return carry
        lax.fori_loop(0, NWAVE // 2, body, 0)

    table_pass(tbl_hbm, 0, True)
    pltpu.sync_copy(out_v, out_hbm.at[wid])


def _make_sc_gather():
    mesh = plsc.VectorSubcoreMesh(core_axis_name="c", subcore_axis_name="s")
    return pl.kernel(
        _sc_body,
        mesh=mesh,
        compiler_params=pltpu.CompilerParams(needs_layout_passes=False),
        out_type=jax.ShapeDtypeStruct((NW, 128, 128), jnp.float32),
        scratch_types=[
            pltpu.VMEM((4, 128), jnp.int32),
            pltpu.VMEM((2, WSZ, 8, EMBED), jnp.float32),
            pltpu.VMEM((128, 128), jnp.float32),
            pltpu.SemaphoreType.DMA,
            pltpu.SemaphoreType.DMA,
        ],
    )


def _sc_body_native(tblT_hbm, idx_hbm, out_hbm, idx_v, ring_v, out_v, sem0, sem1):
    sems = (sem0, sem1)
    wid = lax.axis_index("s") * 2 + lax.axis_index("c")
    pltpu.sync_copy(idx_hbm.at[wid], idx_v)
    lane = jnp.arange(16, dtype=jnp.int32)
    WV = 8

    def scal(g):
        iv16 = idx_v[g >> 7, pl.ds(((g >> 4) & 7) * 16, 16)]
        return jnp.sum(jnp.where(lane == (g & 15), iv16, 0))

    def start(w, s):
        for jj in range(WV):
            g = w * WV + jj
            sj = scal(g)
            off = pl.multiple_of((sj >> 7) * 128, 128)
            pltpu.make_async_copy(
                tblT_hbm.at[:, pl.ds(off, 128)], ring_v.at[s, jj], sems[s]).start()

    def drain(s):
        for _ in range(WV):
            pltpu.make_async_copy(
                tblT_hbm.at[:, pl.ds(0, 128)], ring_v.at[s, 0], sems[s]).wait()

    def extract(w, s):
        for jj in range(WV):
            g = w * WV + jj
            sj = scal(g)
            li = sj & 127
            for h in range(2):
                cvec = lane + h * 16
                vals = plsc.load_gather(ring_v.at[s, jj], [cvec, cvec * 0 + li])
                e0 = g * EMBED + h * 16
                out_v[e0 >> 7, pl.ds((e0 & 127), 16)] = vals

    NWV = 512 // WV

    def body(p, carry):
        w0 = 2 * p
        start(w0 + 1, 1)
        drain(0)
        extract(w0, 0)

        @pl.when(p < NWV // 2 - 1)
        def _():
            start(w0 + 2, 0)
        drain(1)
        extract(w0 + 1, 1)
        return carry

    start(0, 0)
    lax.fori_loop(0, NWV // 2, body, 0)
    pltpu.sync_copy(out_v, out_hbm.at[wid])


def _make_sc_gather_native():
    mesh = plsc.VectorSubcoreMesh(core_axis_name="c", subcore_axis_name="s")
    return pl.kernel(
        _sc_body_native,
        mesh=mesh,
        compiler_params=pltpu.CompilerParams(needs_layout_passes=False),
        out_type=jax.ShapeDtypeStruct((NW, 128, 128), jnp.float32),
        scratch_types=[
            pltpu.VMEM((4, 128), jnp.int32),
            pltpu.VMEM((2, 8, EMBED, 128), jnp.float32),
            pltpu.VMEM((128, 128), jnp.float32),
            pltpu.SemaphoreType.DMA,
            pltpu.SemaphoreType.DMA,
        ],
    )


def _mlp_body(u4_ref, m4_ref, W1u_ref, W1m_ref, b1_ref, W2_ref, b2_ref,
              W3_ref, b3_ref, o_ref):
    x = (jnp.dot(u4_ref[...], W1u_ref[...], preferred_element_type=jnp.float32)
         + jnp.dot(m4_ref[...], W1m_ref[...], preferred_element_type=jnp.float32)
         + b1_ref[...])
    x = jnp.where(x >= 0, x, 0.01 * x)
    x = jnp.dot(x, W2_ref[...], preferred_element_type=jnp.float32) + b2_ref[...]
    x = jnp.where(x >= 0, x, 0.01 * x)
    o_ref[...] = jnp.dot(x, W3_ref[...], preferred_element_type=jnp.float32) + b3_ref[...]


def _mlp(xu, xm, W1u4, W1m4, b14, W24, b24, W34, b34):
    grid = (8,)
    return pl.pallas_call(
        _mlp_body,
        grid=grid,
        in_specs=[
            pl.BlockSpec((512, 128), lambda i: (i, 0)),
            pl.BlockSpec((512, 128), lambda i: (i, 0)),
            pl.BlockSpec((128, 512), lambda i: (0, 0)),
            pl.BlockSpec((128, 512), lambda i: (0, 0)),
            pl.BlockSpec((1, 512), lambda i: (0, 0)),
            pl.BlockSpec((512, 1024), lambda i: (0, 0)),
            pl.BlockSpec((1, 1024), lambda i: (0, 0)),
            pl.BlockSpec((1024, 4), lambda i: (0, 0)),
            pl.BlockSpec((1, 4), lambda i: (0, 0)),
        ],
        out_specs=pl.BlockSpec((512, 4), lambda i: (i, 0)),
        out_shape=jax.ShapeDtypeStruct((4096, 4), jnp.float32),
    )(xu, xm, W1u4, W1m4, b14.reshape(1, -1), W24, b24.reshape(1, -1),
      W34, b34.reshape(1, -1))


def kernel(user, movie, user_table, movie_table, W1, b1, W2, b2, W3, b3):
    B = user.shape[0]
    uidx = user.astype(jnp.int32).reshape(NW, 4, 128)
    midx = movie.astype(jnp.int32).reshape(NW, 4, 128)
    pm = _make_sc_gather()(movie_table, midx)
    pu = _make_sc_gather_native()(user_table.T, uidx)
    xu = pu.reshape(NW * 128, 128)
    xm = pm.reshape(NW * 128, 128)
    W1u4 = _block_diag(*([W1[:EMBED, :]] * 4))
    W1m4 = _block_diag(*([W1[EMBED:, :]] * 4))
    W24 = _block_diag(*([W2] * 4))
    W34 = _block_diag(*([W3] * 4))
    b14 = jnp.tile(b1, 4)
    b24 = jnp.tile(b2, 4)
    b34 = jnp.tile(b3, 4)
    out4 = _mlp(xu, xm, W1u4, W1m4, b14, W24, b24, W34, b34)
    return out4.reshape(B, 1)


# speedup vs baseline: 2.8479x; 1.0578x over previous
import functools
import jax
import jax.numpy as jnp
from jax import lax
from jax.scipy.linalg import block_diag as _block_diag
from jax.experimental import pallas as pl
from jax.experimental.pallas import tpu as pltpu
from jax.experimental.pallas import tpu_sc as plsc

EMBED = 32
NW = 32
WSZ = 32
NWAVE = 16


def _sc_body(tbl_hbm, idx_hbm, out_hbm, idx_v, ring_v, out_v, sem0, sem1):
    sems = (sem0, sem1)
    wid = lax.axis_index("s") * 2 + lax.axis_index("c")
    pltpu.sync_copy(idx_hbm.at[wid], idx_v)
    lane = jnp.arange(16, dtype=jnp.int32)

    def wave_idx(w, h):
        return idx_v[w // 4, pl.ds((w % 4) * WSZ + h * 16, 16)]

    def start(tbl, w, s):
        for h in range(2):
            iv = wave_idx(w, h)
            for j in range(16):
                sj = jnp.sum(jnp.where(lane == j, iv, 0))
                base8 = pl.multiple_of((sj >> 3) * 8, 8)
                pltpu.make_async_copy(
                    tbl.at[pl.ds(base8, 8), :], ring_v.at[s, h * 16 + j],
                    sems[s]).start()

    def drain(tbl, s):
        for _ in range(WSZ):
            pltpu.make_async_copy(
                tbl.at[pl.ds(0, 8), :], ring_v.at[s, 0], sems[s]).wait()

    def extract(w, s):
        ivs = [wave_idx(w, h) for h in range(2)]
        slab = ring_v.at[s]

        def ebody(ci, carry):
            for h in range(2):
                iv7 = ivs[h] & 7
                vals = plsc.load_gather(
                    slab, [lane + h * 16, iv7, iv7 * 0 + ci])
                e = (w * WSZ + h * 16 + lane) * EMBED + ci
                plsc.store_scatter(out_v, [e >> 7, e & 127], vals)
            return carry
        lax.fori_loop(0, EMBED, ebody, 0)

    def table_pass(tbl, woff, guard_last):
        start(tbl, woff, 0)

        def body(p, carry):
            w0 = woff + 2 * p
            start(tbl, w0 + 1, 1)
            drain(tbl, 0)
            extract(w0, 0)

            @pl.when(p < NWAVE // 2 - 1)
            def _():
                start(tbl, w0 + 2, 0)
            drain(tbl, 1)
            extract(w0 + 1, 1)
            return carry
        lax.fori_loop(0, NWAVE // 2, body, 0)

    table_pass(tbl_hbm, 0, True)
    pltpu.sync_copy(out_v, out_hbm.at[wid])


def _make_sc_gather():
    mesh = plsc.VectorSubcoreMesh(core_axis_name="c", subcore_axis_name="s")
    return pl.kernel(
        _sc_body,
        mesh=mesh,
        compiler_params=pltpu.CompilerParams(needs_layout_passes=False),
        out_type=jax.ShapeDtypeStruct((NW, 128, 128), jnp.float32),
        scratch_types=[
            pltpu.VMEM((4, 128), jnp.int32),
            pltpu.VMEM((2, WSZ, 8, EMBED), jnp.float32),
            pltpu.VMEM((128, 128), jnp.float32),
            pltpu.SemaphoreType.DMA,
            pltpu.SemaphoreType.DMA,
        ],
    )


def _sc_body_native(tblT_hbm, idx_hbm, out_hbm, idx_v, ring_v, out_v, sem0, sem1):
    sems = (sem0, sem1)
    wid = lax.axis_index("s") * 2 + lax.axis_index("c")
    pltpu.sync_copy(idx_hbm.at[wid], idx_v)
    lane = jnp.arange(16, dtype=jnp.int32)
    WV = 8

    def scal(g):
        iv16 = idx_v[g >> 7, pl.ds(((g >> 4) & 7) * 16, 16)]
        return jnp.sum(jnp.where(lane == (g & 15), iv16, 0))

    def start(w, s):
        for jj in range(WV):
            g = w * WV + jj
            sj = scal(g)
            off = pl.multiple_of((sj >> 7) * 128, 128)
            pltpu.make_async_copy(
                tblT_hbm.at[:, pl.ds(off, 128)], ring_v.at[s, jj], sems[s]).start()

    def drain(s):
        for _ in range(WV):
            pltpu.make_async_copy(
                tblT_hbm.at[:, pl.ds(0, 128)], ring_v.at[s, 0], sems[s]).wait()

    def extract(w, s):
        for jj in range(WV):
            g = w * WV + jj
            sj = scal(g)
            li = sj & 127
            for h in range(2):
                cvec = lane + h * 16
                vals = plsc.load_gather(ring_v.at[s, jj], [cvec, cvec * 0 + li])
                e0 = g * EMBED + h * 16
                out_v[e0 >> 7, pl.ds((e0 & 127), 16)] = vals

    NWV = 512 // WV

    def body(p, carry):
        w0 = 2 * p
        start(w0 + 1, 1)
        drain(0)
        extract(w0, 0)

        @pl.when(p < NWV // 2 - 1)
        def _():
            start(w0 + 2, 0)
        drain(1)
        extract(w0 + 1, 1)
        return carry

    start(0, 0)
    lax.fori_loop(0, NWV // 2, body, 0)
    pltpu.sync_copy(out_v, out_hbm.at[wid])


def _make_sc_gather_native():
    mesh = plsc.VectorSubcoreMesh(core_axis_name="c", subcore_axis_name="s")
    return pl.kernel(
        _sc_body_native,
        mesh=mesh,
        compiler_params=pltpu.CompilerParams(needs_layout_passes=False),
        out_type=jax.ShapeDtypeStruct((NW, 128, 128), jnp.float32),
        scratch_types=[
            pltpu.VMEM((4, 128), jnp.int32),
            pltpu.VMEM((2, 8, EMBED, 128), jnp.float32),
            pltpu.VMEM((128, 128), jnp.float32),
            pltpu.SemaphoreType.DMA,
            pltpu.SemaphoreType.DMA,
        ],
    )


def _mlp_body(u4_ref, m4_ref, W1u_ref, W1m_ref, b1_ref, W2_ref, b2_ref,
              W3_ref, b3_ref, o_ref):
    x = (jnp.dot(u4_ref[...], W1u_ref[...], preferred_element_type=jnp.float32)
         + jnp.dot(m4_ref[...], W1m_ref[...], preferred_element_type=jnp.float32)
         + b1_ref[...])
    x = jnp.where(x >= 0, x, 0.01 * x)
    x = jnp.dot(x, W2_ref[...], preferred_element_type=jnp.float32) + b2_ref[...]
    x = jnp.where(x >= 0, x, 0.01 * x)
    o_ref[...] = jnp.dot(x, W3_ref[...], preferred_element_type=jnp.float32) + b3_ref[...]


def _mlp(xu, xm, W1u4, W1m4, b14, W24, b24, W34, b34):
    grid = (8,)
    return pl.pallas_call(
        _mlp_body,
        grid=grid,
        in_specs=[
            pl.BlockSpec((512, 128), lambda i: (i, 0)),
            pl.BlockSpec((512, 128), lambda i: (i, 0)),
            pl.BlockSpec((128, 512), lambda i: (0, 0)),
            pl.BlockSpec((128, 512), lambda i: (0, 0)),
            pl.BlockSpec((1, 512), lambda i: (0, 0)),
            pl.BlockSpec((512, 1024), lambda i: (0, 0)),
            pl.BlockSpec((1, 1024), lambda i: (0, 0)),
            pl.BlockSpec((1024, 4), lambda i: (0, 0)),
            pl.BlockSpec((1, 4), lambda i: (0, 0)),
        ],
        out_specs=pl.BlockSpec((512, 4), lambda i: (i, 0)),
        out_shape=jax.ShapeDtypeStruct((4096, 4), jnp.float32),
    )(xu, xm, W1u4, W1m4, b14.reshape(1, -1), W24, b24.reshape(1, -1),
      W34, b34.reshape(1, -1))


def kernel(user, movie, user_table, movie_table, W1, b1, W2, b2, W3, b3):
    B = user.shape[0]
    uidx = user.astype(jnp.int32).reshape(NW, 4, 128)
    midx = movie.astype(jnp.int32).reshape(NW, 4, 128)
    pu = _make_sc_gather_native()(user_table.T, uidx)
    midx2, pu = lax.optimization_barrier((midx, pu))
    pm = _make_sc_gather()(movie_table, midx2)
    xu = pu.reshape(NW * 128, 128)
    xm = pm.reshape(NW * 128, 128)
    W1u4 = _block_diag(*([W1[:EMBED, :]] * 4))
    W1m4 = _block_diag(*([W1[EMBED:, :]] * 4))
    W24 = _block_diag(*([W2] * 4))
    W34 = _block_diag(*([W3] * 4))
    b14 = jnp.tile(b1, 4)
    b24 = jnp.tile(b2, 4)
    b34 = jnp.tile(b3, 4)
    out4 = _mlp(xu, xm, W1u4, W1m4, b14, W24, b24, W34, b34)
    return out4.reshape(B, 1)


# speedup vs baseline: 2.8584x; 1.0037x over previous
import jax
import jax.numpy as jnp
from jax import lax
from jax.scipy.linalg import block_diag as _block_diag
from jax.experimental import pallas as pl
from jax.experimental.pallas import tpu as pltpu
from jax.experimental.pallas import tpu_sc as plsc

EMBED = 32
NW = 32
WSZ = 32
NWAVE = 16


def _sc_body(tbl_hbm, idx_hbm, out_hbm, idx_v, ring_v, out_v, sem0, sem1):
    sems = (sem0, sem1)
    wid = lax.axis_index("s") * 2 + lax.axis_index("c")
    pltpu.sync_copy(idx_hbm.at[wid], idx_v)
    lane = jnp.arange(16, dtype=jnp.int32)

    def wave_idx(w, h):
        return idx_v[w // 4, pl.ds((w % 4) * WSZ + h * 16, 16)]

    def start(tbl, w, s):
        for h in range(2):
            iv = wave_idx(w, h)
            for j in range(16):
                sj = jnp.sum(jnp.where(lane == j, iv, 0))
                base8 = pl.multiple_of((sj >> 3) * 8, 8)
                pltpu.make_async_copy(
                    tbl.at[pl.ds(base8, 8), :], ring_v.at[s, h * 16 + j],
                    sems[s]).start()

    def drain(tbl, s):
        for _ in range(WSZ):
            pltpu.make_async_copy(
                tbl.at[pl.ds(0, 8), :], ring_v.at[s, 0], sems[s]).wait()

    def extract(w, s):
        ivs = [wave_idx(w, h) for h in range(2)]
        slab = ring_v.at[s]

        def ebody(ci, carry):
            for h in range(2):
                iv7 = ivs[h] & 7
                vals = plsc.load_gather(
                    slab, [lane + h * 16, iv7, iv7 * 0 + ci])
                e = (w * WSZ + h * 16 + lane) * EMBED + ci
                plsc.store_scatter(out_v, [e >> 7, e & 127], vals)
            return carry
        lax.fori_loop(0, EMBED, ebody, 0)

    def table_pass(tbl, woff):
        start(tbl, woff, 0)

        def body(p, carry):
            w0 = woff + 2 * p
            start(tbl, w0 + 1, 1)
            drain(tbl, 0)
            extract(w0, 0)

            @pl.when(p < NWAVE // 2 - 1)
            def _():
                start(tbl, w0 + 2, 0)
            drain(tbl, 1)
            extract(w0 + 1, 1)
            return carry
        lax.fori_loop(0, NWAVE // 2, body, 0)

    table_pass(tbl_hbm, 0)
    pltpu.sync_copy(out_v, out_hbm.at[wid])


def _make_sc_gather():
    mesh = plsc.VectorSubcoreMesh(core_axis_name="c", subcore_axis_name="s")
    return pl.kernel(
        _sc_body,
        mesh=mesh,
        compiler_params=pltpu.CompilerParams(needs_layout_passes=False),
        out_type=jax.ShapeDtypeStruct((NW, 128, 128), jnp.float32),
        scratch_types=[
            pltpu.VMEM((4, 128), jnp.int32),
            pltpu.VMEM((2, WSZ, 8, EMBED), jnp.float32),
            pltpu.VMEM((128, 128), jnp.float32),
            pltpu.SemaphoreType.DMA,
            pltpu.SemaphoreType.DMA,
        ],
    )


def _sc_body_native(tblT_hbm, idx_hbm, out_hbm, idx_v, ring_v, out_v, sem0, sem1):
    sems = (sem0, sem1)
    wid = lax.axis_index("s") * 2 + lax.axis_index("c")
    pltpu.sync_copy(idx_hbm.at[wid], idx_v)
    lane = jnp.arange(16, dtype=jnp.int32)
    WV = 8

    def scal(g):
        iv16 = idx_v[g >> 7, pl.ds(((g >> 4) & 7) * 16, 16)]
        return jnp.sum(jnp.where(lane == (g & 15), iv16, 0))

    def start(w, s):
        for jj in range(WV):
            g = w * WV + jj
            sj = scal(g)
            off = pl.multiple_of((sj >> 7) * 128, 128)
            pltpu.make_async_copy(
                tblT_hbm.at[:, pl.ds(off, 128)], ring_v.at[s, jj], sems[s]).start()

    def drain(s):
        for _ in range(WV):
            pltpu.make_async_copy(
                tblT_hbm.at[:, pl.ds(0, 128)], ring_v.at[s, 0], sems[s]).wait()

    def extract(w, s):
        for jj in range(WV):
            g = w * WV + jj
            sj = scal(g)
            li = sj & 127
            for h in range(2):
                cvec = lane + h * 16
                vals = plsc.load_gather(ring_v.at[s, jj], [cvec, cvec * 0 + li])
                e0 = g * EMBED + h * 16
                out_v[e0 >> 7, pl.ds((e0 & 127), 16)] = vals

    NWV = 512 // WV

    def body(p, carry):
        w0 = 2 * p
        start(w0 + 1, 1)
        drain(0)
        extract(w0, 0)

        @pl.when(p < NWV // 2 - 1)
        def _():
            start(w0 + 2, 0)
        drain(1)
        extract(w0 + 1, 1)
        return carry

    start(0, 0)
    lax.fori_loop(0, NWV // 2, body, 0)
    pltpu.sync_copy(out_v, out_hbm.at[wid])


def _make_sc_gather_native():
    mesh = plsc.VectorSubcoreMesh(core_axis_name="c", subcore_axis_name="s")
    return pl.kernel(
        _sc_body_native,
        mesh=mesh,
        compiler_params=pltpu.CompilerParams(needs_layout_passes=False),
        out_type=jax.ShapeDtypeStruct((NW, 128, 128), jnp.float32),
        scratch_types=[
            pltpu.VMEM((4, 128), jnp.int32),
            pltpu.VMEM((2, 8, EMBED, 128), jnp.float32),
            pltpu.VMEM((128, 128), jnp.float32),
            pltpu.SemaphoreType.DMA,
            pltpu.SemaphoreType.DMA,
        ],
    )


def _mlp_body(u4_ref, m4_ref, W1u_ref, W1m_ref, b1_ref, W2_ref, b2_ref,
              W3_ref, b3_ref, o_ref):
    x = (jnp.dot(u4_ref[...], W1u_ref[...], preferred_element_type=jnp.float32)
         + jnp.dot(m4_ref[...], W1m_ref[...], preferred_element_type=jnp.float32)
         + b1_ref[...])
    x = jnp.where(x >= 0, x, 0.01 * x)
    x = jnp.dot(x, W2_ref[...], preferred_element_type=jnp.float32) + b2_ref[...]
    x = jnp.where(x >= 0, x, 0.01 * x)
    o_ref[...] = jnp.dot(x, W3_ref[...], preferred_element_type=jnp.float32) + b3_ref[...]


def _mlp(xu, xm, W1u4, W1m4, b14, W24, b24, W34, b34):
    grid = (8,)
    return pl.pallas_call(
        _mlp_body,
        grid=grid,
        in_specs=[
            pl.BlockSpec((512, 128), lambda i: (i, 0)),
            pl.BlockSpec((512, 128), lambda i: (i, 0)),
            pl.BlockSpec((128, 512), lambda i: (0, 0)),
            pl.BlockSpec((128, 512), lambda i: (0, 0)),
            pl.BlockSpec((1, 512), lambda i: (0, 0)),
            pl.BlockSpec((512, 1024), lambda i: (0, 0)),
            pl.BlockSpec((1, 1024), lambda i: (0, 0)),
            pl.BlockSpec((1024, 4), lambda i: (0, 0)),
            pl.BlockSpec((1, 4), lambda i: (0, 0)),
        ],
        out_specs=pl.BlockSpec((512, 4), lambda i: (i, 0)),
        out_shape=jax.ShapeDtypeStruct((4096, 4), jnp.float32),
    )(xu, xm, W1u4, W1m4, b14.reshape(1, -1), W24, b24.reshape(1, -1),
      W34, b34.reshape(1, -1))


def kernel(user, movie, user_table, movie_table, W1, b1, W2, b2, W3, b3):
    B = user.shape[0]
    uidx = user.astype(jnp.int32).reshape(NW, 4, 128)
    midx = movie.astype(jnp.int32).reshape(NW, 4, 128)
    pu = _make_sc_gather_native()(user_table.T, uidx)
    midx2, pu = lax.optimization_barrier((midx, pu))
    pm = _make_sc_gather()(movie_table, midx2)
    xu = pu.reshape(NW * 128, 128)
    xm = pm.reshape(NW * 128, 128)
    W1u4 = _block_diag(*([W1[:EMBED, :]] * 4))
    W1m4 = _block_diag(*([W1[EMBED:, :]] * 4))
    W24 = _block_diag(*([W2] * 4))
    W34 = _block_diag(*([W3] * 4))
    b14 = jnp.tile(b1, 4)
    b24 = jnp.tile(b2, 4)
    b34 = jnp.tile(b3, 4)
    out4 = _mlp(xu, xm, W1u4, W1m4, b14, W24, b24, W34, b34)
    return out4.reshape(B, 1)
